# baseline (speedup 1.0000x reference)
import jax
import jax.numpy as jnp
from jax import lax
from jax.experimental import pallas as pl
from jax.experimental.pallas import tpu as pltpu
from jax.experimental.pallas import tpu_sc as plsc

HEAD_DIM = 64
Q_LEN = 32
K_LEN = 8192
WIN = 544
TAIL = K_LEN - WIN
BUF_ROWS = 1024
PAD_ROWS = 2048
TAB_BASE = 480
TAB_ROWS = 1056 - TAB_BASE


def _sc_body(padded_hbm, out_hbm, tab, buf, sem):
    c = lax.axis_index("c")
    s = lax.axis_index("s")
    q = s * 2 + c

    r_tab = pltpu.async_copy(
        padded_hbm.at[pl.ds(TAB_BASE * HEAD_DIM, TAB_ROWS * HEAD_DIM)], tab, sem)
    r_buf = pltpu.async_copy(
        padded_hbm.at[pl.ds(1024 * HEAD_DIM, BUF_ROWS * HEAD_DIM)], buf, sem)
    r_tab.wait()
    r_buf.wait()

    out_base = q * (K_LEN * HEAD_DIM)
    n_full = TAIL // BUF_ROWS
    rem = TAIL - n_full * BUF_ROWS
    handles = [pltpu.async_copy(
        tab.at[pl.ds((512 - TAB_BASE - q) * HEAD_DIM, WIN * HEAD_DIM)],
        out_hbm.at[pl.ds(out_base, WIN * HEAD_DIM)], sem)]
    for j in range(n_full):
        handles.append(pltpu.async_copy(
            buf,
            out_hbm.at[pl.ds(out_base + (WIN + j * BUF_ROWS) * HEAD_DIM,
                             BUF_ROWS * HEAD_DIM)], sem))
    handles.append(pltpu.async_copy(
        buf.at[pl.ds(0, rem * HEAD_DIM)],
        out_hbm.at[pl.ds(out_base + (WIN + n_full * BUF_ROWS) * HEAD_DIM,
                         rem * HEAD_DIM)], sem))
    for h in handles:
        h.wait()


def kernel(query_len, key_len, pos_embed):
    del query_len, key_len
    pad = jnp.broadcast_to(pos_embed[-1], (PAD_ROWS - 1025, HEAD_DIM))
    padded = jnp.concatenate([pos_embed, pad], axis=0).reshape(-1)
    mesh = plsc.VectorSubcoreMesh(core_axis_name="c", subcore_axis_name="s")
    f = pl.kernel(
        _sc_body,
        out_type=jax.ShapeDtypeStruct((Q_LEN * K_LEN * HEAD_DIM,), jnp.float32),
        mesh=mesh,
        scratch_types=[
            pltpu.VMEM((TAB_ROWS * HEAD_DIM,), jnp.float32),
            pltpu.VMEM((BUF_ROWS * HEAD_DIM,), jnp.float32),
            pltpu.SemaphoreType.DMA,
        ],
    )
    return f(padded).reshape(Q_LEN, K_LEN, HEAD_DIM)


# speedup vs baseline: 1.2322x; 1.2322x over previous
import jax
import jax.numpy as jnp
from jax import lax
from jax.experimental import pallas as pl
from jax.experimental.pallas import tpu as pltpu
from jax.experimental.pallas import tpu_sc as plsc

HEAD_DIM = 64
Q_LEN = 32
K_LEN = 8192
WIN = 544
TAIL = K_LEN - WIN
BUF_ROWS = 256
PAD_ROWS = 1280
TAB_BASE = 480
TAB_ROWS = 1056 - TAB_BASE


def _sc_body(padded_hbm, out_hbm, tab, buf, sem):
    c = lax.axis_index("c")
    s = lax.axis_index("s")
    q = s * 2 + c

    r_tab = pltpu.async_copy(padded_hbm.at[pl.ds(TAB_BASE, TAB_ROWS)], tab, sem)
    r_buf = pltpu.async_copy(padded_hbm.at[pl.ds(1024, BUF_ROWS)], buf, sem)
    r_tab.wait()
    r_buf.wait()

    n_full = TAIL // BUF_ROWS
    rem = TAIL - n_full * BUF_ROWS
    handles = [pltpu.async_copy(tab.at[pl.ds(512 - TAB_BASE - q, WIN)],
                                out_hbm.at[q, pl.ds(0, WIN)], sem)]
    for j in range(n_full):
        handles.append(pltpu.async_copy(
            buf, out_hbm.at[q, pl.ds(WIN + j * BUF_ROWS, BUF_ROWS)], sem))
    handles.append(pltpu.async_copy(
        buf.at[pl.ds(0, rem)],
        out_hbm.at[q, pl.ds(WIN + n_full * BUF_ROWS, rem)], sem))
    for h in handles:
        h.wait()


def kernel(query_len, key_len, pos_embed):
    del query_len, key_len
    pad = jnp.broadcast_to(pos_embed[-1], (PAD_ROWS - 1025, HEAD_DIM))
    padded = jnp.concatenate([pos_embed, pad], axis=0)
    mesh = plsc.VectorSubcoreMesh(core_axis_name="c", subcore_axis_name="s")
    f = pl.kernel(
        _sc_body,
        out_type=jax.ShapeDtypeStruct((Q_LEN, K_LEN, HEAD_DIM), jnp.float32),
        mesh=mesh,
        scratch_types=[
            pltpu.VMEM((TAB_ROWS, HEAD_DIM), jnp.float32),
            pltpu.VMEM((BUF_ROWS, HEAD_DIM), jnp.float32),
            pltpu.SemaphoreType.DMA,
        ],
        compiler_params=pltpu.CompilerParams(use_tc_tiling_on_sc=True),
    )
    return f(padded)
